# baseline (speedup 1.0000x reference)
import functools

import jax
import jax.numpy as jnp
from jax import lax
from jax.experimental import pallas as pl
from jax.experimental.pallas import tpu as pltpu
from jax.experimental.pallas import tpu_sc as plsc

_INFO = plsc.get_sparse_core_info()
_NC = _INFO.num_cores
_NS = _INFO.num_subcores
_NW = _NC * _NS

_B = 16384
_D = 128
_B_PER_W = _B // _NW
_CHUNK = 64
_NCHUNK = _B_PER_W // _CHUNK


@functools.partial(
    pl.kernel,
    mesh=plsc.VectorSubcoreMesh(core_axis_name="c", subcore_axis_name="s"),
    out_type=jax.ShapeDtypeStruct((_B, _D), jnp.float32),
    scratch_types=[
        pltpu.VMEM((_NCHUNK, _CHUNK), jnp.int32),
        pltpu.VMEM((_B_PER_W, _D), jnp.float32),
    ]
    + [pltpu.SemaphoreType.DMA] * (2 * _NCHUNK),
)
def _gather_rows(table_hbm, idx_hbm, out_hbm, idx_v, rows_v, *sems):
    gsem, wsem = sems[:_NCHUNK], sems[_NCHUNK:]
    wid = lax.axis_index("s") * _NC + lax.axis_index("c")
    base = wid * _B_PER_W
    pltpu.sync_copy(idx_hbm.at[pl.ds(wid * _NCHUNK, _NCHUNK)], idx_v)
    gathers = [
        pltpu.async_copy(
            table_hbm.at[idx_v.at[j]],
            rows_v.at[pl.ds(j * _CHUNK, _CHUNK)],
            gsem[j],
        )
        for j in range(_NCHUNK)
    ]
    writes = []
    for j in range(_NCHUNK):
        gathers[j].wait()
        writes.append(
            pltpu.async_copy(
                rows_v.at[pl.ds(j * _CHUNK, _CHUNK)],
                out_hbm.at[pl.ds(base + j * _CHUNK, _CHUNK)],
                wsem[j],
            )
        )
    for w in writes:
        w.wait()


@jax.jit
def kernel(input_tensor, indices):
    idx2d = indices.reshape(_NW * _NCHUNK, _CHUNK).astype(jnp.int32)
    return _gather_rows(input_tensor, idx2d)


# speedup vs baseline: 1.0321x; 1.0321x over previous
import functools

import jax
import jax.numpy as jnp
from jax import lax
from jax.experimental import pallas as pl
from jax.experimental.pallas import tpu as pltpu
from jax.experimental.pallas import tpu_sc as plsc

_INFO = plsc.get_sparse_core_info()
_NC = _INFO.num_cores
_NS = _INFO.num_subcores
_NW = _NC * _NS

_B = 16384
_D = 128
_B_PER_W = _B // _NW
_CHUNK = 128
_NCHUNK = _B_PER_W // _CHUNK


@functools.partial(
    pl.kernel,
    mesh=plsc.VectorSubcoreMesh(core_axis_name="c", subcore_axis_name="s"),
    out_type=jax.ShapeDtypeStruct((_B, _D), jnp.float32),
    scratch_types=[
        pltpu.VMEM((_NCHUNK, _CHUNK), jnp.int32),
        pltpu.VMEM((_B_PER_W, _D), jnp.float32),
    ]
    + [pltpu.SemaphoreType.DMA] * 3,
)
def _gather_rows(table_hbm, idx_hbm, out_hbm, idx_v, rows_v, sem_a, sem_b, sem_w):
    half = _B_PER_W // 2
    wid = lax.axis_index("s") * _NC + lax.axis_index("c")
    base = wid * _B_PER_W
    pltpu.sync_copy(idx_hbm.at[pl.ds(wid * _NCHUNK, _NCHUNK)], idx_v)
    gathers = [
        pltpu.async_copy(
            table_hbm.at[idx_v.at[j]],
            rows_v.at[pl.ds(j * _CHUNK, _CHUNK)],
            sem_a if j < _NCHUNK // 2 else sem_b,
        )
        for j in range(_NCHUNK)
    ]
    for c in gathers[: _NCHUNK // 2]:
        c.wait()
    w = pltpu.async_copy(
        rows_v.at[pl.ds(0, half)], out_hbm.at[pl.ds(base, half)], sem_w
    )
    for c in gathers[_NCHUNK // 2 :]:
        c.wait()
    pltpu.sync_copy(
        rows_v.at[pl.ds(half, half)], out_hbm.at[pl.ds(base + half, half)]
    )
    w.wait()


@jax.jit
def kernel(input_tensor, indices):
    idx2d = indices.reshape(_NW * _NCHUNK, _CHUNK).astype(jnp.int32)
    return _gather_rows(input_tensor, idx2d)
